# baseline (speedup 1.0000x reference)
import functools

import jax
import jax.numpy as jnp
from jax import lax
from jax.experimental import pallas as pl
from jax.experimental.pallas import tpu as pltpu
from jax.experimental.pallas import tpu_sc as plsc
from jax._src.pallas import mpmd as _mpmd


def _copy_body(x_ref, o_ref):
    o_ref[...] = x_ref[...]


def _tc_copy(x2d, blk):
    r, d = x2d.shape
    return pl.pallas_call(
        _copy_body,
        grid=(r // blk,),
        in_specs=[pl.BlockSpec((blk, d), lambda i: (i, 0))],
        out_specs=pl.BlockSpec((blk, d), lambda i: (i, 0)),
        out_shape=jax.ShapeDtypeStruct(x2d.shape, x2d.dtype),
    )(x2d)


def _make_sc_scatter(b, s, d, n, chunk=8):
    nc, ns = 2, 16
    mesh = plsc.VectorSubcoreMesh(
        core_axis_name="c", subcore_axis_name="s", num_cores=nc, num_subcores=ns
    )
    ntasks = b * n
    nworkers = ntasks // chunk

    def sc_scatter(y_in, vals_hbm, flat_hbm, out_ref, idx_v, rows_v, sem):
        del y_in
        wid = lax.axis_index("s") * nc + lax.axis_index("c")

        @pl.when(wid < nworkers)
        def _():
            t0 = wid * chunk
            i0 = t0 % n
            vals_cp = pltpu.make_async_copy(
                vals_hbm.at[pl.ds(i0, chunk)], rows_v, sem
            )
            vals_cp.start()
            pltpu.sync_copy(flat_hbm.at[pl.ds(t0, chunk)], idx_v)
            vals_cp.wait()
            pltpu.async_copy(rows_v, out_ref.at[idx_v], sem).wait()

    return _mpmd._mpmd_map(
        [(mesh, sc_scatter)],
        jax.ShapeDtypeStruct((b * s, d), jnp.float32),
        input_output_aliases={0: 0},
        scratch_types=[
            pltpu.VMEM((chunk,), jnp.int32),
            pltpu.VMEM((chunk, d), jnp.float32),
            pltpu.SemaphoreType.DMA,
        ],
    )


def kernel(x, replace_vals, replace_idx):
    b, s, d = x.shape
    n = replace_vals.shape[0]
    x2d = x.reshape(b * s, d)
    y = _tc_copy(x2d, blk=512)
    flat_idx = (
        replace_idx[None, :] + (jnp.arange(b, dtype=jnp.int32) * s)[:, None]
    ).reshape(-1)
    out = _make_sc_scatter(b, s, d, n)(y, replace_vals, flat_idx)
    return out.reshape(b, s, d)


# speedup vs baseline: 1.0008x; 1.0008x over previous
import functools

import jax
import jax.numpy as jnp
from jax import lax
from jax.experimental import pallas as pl
from jax.experimental.pallas import tpu as pltpu
from jax.experimental.pallas import tpu_sc as plsc


def _copy_body(x_ref, o_ref):
    o_ref[...] = x_ref[...]


def _tc_copy(x2d, blk):
    r, d = x2d.shape
    return pl.pallas_call(
        _copy_body,
        grid=(r // blk,),
        in_specs=[pl.BlockSpec((blk, d), lambda i: (i, 0))],
        out_specs=pl.BlockSpec((blk, d), lambda i: (i, 0)),
        out_shape=jax.ShapeDtypeStruct(x2d.shape, x2d.dtype),
    )(x2d)


def _make_sc_scatter(b, s, d, n, chunk=8):
    nc, ns = 2, 16
    mesh = plsc.VectorSubcoreMesh(
        core_axis_name="c", subcore_axis_name="s", num_cores=nc, num_subcores=ns
    )
    ntasks = b * n
    nworkers = ntasks // chunk

    @functools.partial(
        pl.kernel,
        out_type=(),
        mesh=mesh,
        scratch_types=[
            pltpu.VMEM((chunk,), jnp.int32),
            pltpu.VMEM((chunk, d), jnp.float32),
            pltpu.SemaphoreType.DMA,
        ],
    )
    def sc_scatter(out_ref, vals_hbm, flat_hbm, idx_v, rows_v, sem):
        wid = lax.axis_index("s") * nc + lax.axis_index("c")

        @pl.when(wid < nworkers)
        def _():
            t0 = wid * chunk
            i0 = t0 % n
            vals_cp = pltpu.make_async_copy(
                vals_hbm.at[pl.ds(i0, chunk)], rows_v, sem
            )
            vals_cp.start()
            pltpu.sync_copy(flat_hbm.at[pl.ds(t0, chunk)], idx_v)
            vals_cp.wait()
            pltpu.async_copy(rows_v, out_ref.at[idx_v], sem).wait()

    return sc_scatter


def kernel(x, replace_vals, replace_idx):
    b, s, d = x.shape
    n = replace_vals.shape[0]
    x2d = x.reshape(b * s, d)
    y = _tc_copy(x2d, blk=512)
    flat_idx = (
        replace_idx[None, :] + (jnp.arange(b, dtype=jnp.int32) * s)[:, None]
    ).reshape(-1)
    y_ref = jax.new_ref(y)
    _make_sc_scatter(b, s, d, n)(y_ref, replace_vals, flat_idx)
    return jax.freeze(y_ref).reshape(b, s, d)
